# Initial kernel scaffold; baseline (speedup 1.0000x reference)
import jax
import jax.numpy as jnp
from jax.experimental import pallas as pl


def kernel(x, edge_index, W0, b0, W1, b1, Wp, bp, Wv, bv):
    raise NotImplementedError("write your pallas kernel here")


# speedup vs baseline: 11.5611x; 11.5611x over previous
import functools

import jax
import jax.numpy as jnp
from jax import lax
from jax.experimental import pallas as pl
from jax.experimental.pallas import tpu as pltpu
from jax.experimental.pallas import tpu_sc as plsc

_N = 10000
_E = 320000
_D = 128
_NPAD = 10240
_NC = 2
_NS = 16
_NW = _NC * _NS
_RPT = _NPAD // _NS
_CW = 80
_NCH = (_E // _NW) // _CW
_DCH = (_E // _NS) // _CW

_mesh = plsc.VectorSubcoreMesh(core_axis_name="c", subcore_axis_name="s")


@functools.partial(
    pl.kernel,
    out_type=jax.ShapeDtypeStruct((_NPAD,), jnp.float32),
    mesh=_mesh,
    scratch_types=[
        pltpu.VMEM((_DCH, _CW), jnp.int32),
        pltpu.VMEM((_CW,), jnp.float32),
        pltpu.VMEM((_RPT,), jnp.float32),
        pltpu.VMEM_SHARED((_NPAD,), jnp.float32),
    ],
)
def _deg_kernel(src_hbm, inv_hbm, idx_v, ones_v, val_v, deg_sh):
    c = lax.axis_index("c")
    s = lax.axis_index("s")
    pltpu.sync_copy(src_hbm.at[s], idx_v)

    for q in range(_CW // 16):
        ones_v[pl.ds(q * 16, 16)] = jnp.full((16,), 1.0, jnp.float32)

    def _zero_val(j, _):
        val_v[pl.ds(j * 16, 16)] = jnp.zeros((16,), jnp.float32)
        return _

    lax.fori_loop(0, _RPT // 16, _zero_val, None)
    pltpu.sync_copy(val_v, deg_sh.at[pl.ds(s * _RPT, _RPT)])
    plsc.subcore_barrier()
    def _scat(i, _):
        pltpu.sync_copy(ones_v, deg_sh.at[idx_v.at[i]], add=True)
        return _

    lax.fori_loop(0, _DCH, _scat, None)
    plsc.subcore_barrier()
    pltpu.sync_copy(deg_sh.at[pl.ds(s * _RPT, _RPT)], val_v)

    def _inv(j, _):
        v = val_v[pl.ds(j * 16, 16)]
        val_v[pl.ds(j * 16, 16)] = 1.0 / jnp.maximum(v, 1.0)
        return _

    lax.fori_loop(0, _RPT // 16, _inv, None)

    @pl.when(c == 0)
    def _():
        pltpu.sync_copy(val_v, inv_hbm.at[pl.ds(s * _RPT, _RPT)])


@functools.partial(
    pl.kernel,
    out_type=jax.ShapeDtypeStruct((_NC, _NPAD, _D), jnp.float32),
    mesh=_mesh,
    scratch_types=[
        pltpu.VMEM((_NCH, _CW), jnp.int32),
        pltpu.VMEM((_NCH, _CW), jnp.int32),
        pltpu.VMEM((_CW, _D), jnp.float32),
        pltpu.VMEM_SHARED((_NPAD, _D), jnp.float32),
    ],
)
def _agg_kernel(tbl_hbm, src_hbm, dst_hbm, out_hbm, sidx_v, didx_v, buf_v, acc_sh):
    c = lax.axis_index("c")
    s = lax.axis_index("s")
    w = s * _NC + c
    pltpu.sync_copy(src_hbm.at[w], sidx_v)
    pltpu.sync_copy(dst_hbm.at[w], didx_v)

    def _zero_buf(i, _):
        for q in range(_D // 16):
            buf_v[i, pl.ds(q * 16, 16)] = jnp.zeros((16,), jnp.float32)
        return _

    lax.fori_loop(0, _CW, _zero_buf, None)

    def _zero_acc(k, _):
        pltpu.sync_copy(buf_v, acc_sh.at[pl.ds(s * _RPT + k * _CW, _CW)])
        return _

    lax.fori_loop(0, _RPT // _CW, _zero_acc, None)
    plsc.subcore_barrier()

    def _step(i, _):
        pltpu.sync_copy(tbl_hbm.at[sidx_v.at[i]], buf_v)
        pltpu.sync_copy(buf_v, acc_sh.at[didx_v.at[i]], add=True)
        return _

    lax.fori_loop(0, _NCH, _step, None)
    plsc.subcore_barrier()
    pltpu.sync_copy(acc_sh.at[pl.ds(s * _RPT, _RPT)],
                    out_hbm.at[c, pl.ds(s * _RPT, _RPT)])


_BR = 400


def _scale_body(x_ref, inv_ref, o_ref):
    o_ref[...] = x_ref[...] * inv_ref[...]


_scale = pl.pallas_call(
    _scale_body,
    grid=(_N // _BR,),
    in_specs=[
        pl.BlockSpec((_BR, _D), lambda i: (i, 0)),
        pl.BlockSpec((_BR, 1), lambda i: (i, 0)),
    ],
    out_specs=pl.BlockSpec((_BR, _D), lambda i: (i, 0)),
    out_shape=jax.ShapeDtypeStruct((_N, _D), jnp.float32),
)


def _mm1_body(p_ref, w_ref, b_ref, inv_ref, o_ref):
    a = p_ref[0] + p_ref[1]
    h = jnp.dot(a, w_ref[...], preferred_element_type=jnp.float32, precision=lax.Precision.HIGHEST) + b_ref[...]
    o_ref[...] = jnp.maximum(h, 0.0) * inv_ref[...]


_mm1 = pl.pallas_call(
    _mm1_body,
    grid=(_N // _BR,),
    in_specs=[
        pl.BlockSpec((_NC, _BR, _D), lambda i: (0, i, 0)),
        pl.BlockSpec((_D, _D), lambda i: (0, 0)),
        pl.BlockSpec((1, _D), lambda i: (0, 0)),
        pl.BlockSpec((_BR, 1), lambda i: (i, 0)),
    ],
    out_specs=pl.BlockSpec((_BR, _D), lambda i: (i, 0)),
    out_shape=jax.ShapeDtypeStruct((_N, _D), jnp.float32),
)


def _fin_body(p_ref, w1_ref, b1_ref, wp_ref, bp_ref, wv_ref, bv_ref,
              pi_ref, v_ref, acc_ref):
    i = pl.program_id(0)
    a = p_ref[0] + p_ref[1]
    h2 = jnp.dot(a, w1_ref[...], preferred_element_type=jnp.float32,
                 precision=lax.Precision.HIGHEST) + b1_ref[...]
    pi_ref[...] = jnp.dot(h2, wp_ref[...], preferred_element_type=jnp.float32,
                          precision=lax.Precision.HIGHEST) + bp_ref[...]
    colsum = jnp.sum(a, axis=0, keepdims=True)

    @pl.when(i == 0)
    def _():
        acc_ref[...] = colsum

    @pl.when(i > 0)
    def _():
        acc_ref[...] = acc_ref[...] + colsum

    @pl.when(i == _N // _BR - 1)
    def _():
        m = acc_ref[...] * (1.0 / _N)
        h2m = jnp.dot(m, w1_ref[...], preferred_element_type=jnp.float32, precision=lax.Precision.HIGHEST) \
            + b1_ref[...]
        v_ref[...] = jnp.dot(h2m, wv_ref[...],
                             preferred_element_type=jnp.float32, precision=lax.Precision.HIGHEST) + bv_ref[...]


_fin = pl.pallas_call(
    _fin_body,
    grid=(_N // _BR,),
    in_specs=[
        pl.BlockSpec((_NC, _BR, _D), lambda i: (0, i, 0)),
        pl.BlockSpec((_D, _D), lambda i: (0, 0)),
        pl.BlockSpec((1, _D), lambda i: (0, 0)),
        pl.BlockSpec((_D, 1), lambda i: (0, 0)),
        pl.BlockSpec((1, 1), lambda i: (0, 0)),
        pl.BlockSpec((_D, 1), lambda i: (0, 0)),
        pl.BlockSpec((1, 1), lambda i: (0, 0)),
    ],
    out_specs=[
        pl.BlockSpec((_BR, 1), lambda i: (i, 0)),
        pl.BlockSpec((1, 1), lambda i: (0, 0)),
    ],
    out_shape=[
        jax.ShapeDtypeStruct((_N, 1), jnp.float32),
        jax.ShapeDtypeStruct((1, 1), jnp.float32),
    ],
    scratch_shapes=[pltpu.VMEM((1, _D), jnp.float32)],
)


def kernel(x, edge_index, W0, b0, W1, b1, Wp, bp, Wv, bv):
    src = edge_index[0]
    dst = edge_index[1]
    src_deg = src.reshape(_NS, _DCH, _CW)
    src3 = src.reshape(_NW, _NCH, _CW)
    dst3 = dst.reshape(_NW, _NCH, _CW)

    inv_pad = _deg_kernel(src_deg)
    inv2 = inv_pad[:_N].reshape(_N, 1)

    xn = _scale(x, inv2)
    p1 = _agg_kernel(xn, src3, dst3)
    h1n = _mm1(p1, W0, b0.reshape(1, _D), inv2)
    p2 = _agg_kernel(h1n, src3, dst3)
    PI, V = _fin(p2, W1, b1.reshape(1, _D), Wp, bp.reshape(1, 1),
                 Wv, bv.reshape(1, 1))
    return (PI, V)


# speedup vs baseline: 13.2335x; 1.1447x over previous
import functools

import jax
import jax.numpy as jnp
from jax import lax
from jax.experimental import pallas as pl
from jax.experimental.pallas import tpu as pltpu
from jax.experimental.pallas import tpu_sc as plsc

_N = 10000
_E = 320000
_D = 128
_NPAD = 10240
_NC = 2
_NS = 16
_NW = _NC * _NS
_RPT = _NPAD // _NS
_CW = 128
_EPW = _E // _NW
_NCH = -(-_EPW // _CW)
_EPWP = _NCH * _CW
_EPT = _E // _NS
_DCH = -(-_EPT // _CW)
_EPTP = _DCH * _CW

_mesh = plsc.VectorSubcoreMesh(core_axis_name="c", subcore_axis_name="s")


@functools.partial(
    pl.kernel,
    out_type=jax.ShapeDtypeStruct((_NPAD,), jnp.float32),
    mesh=_mesh,
    scratch_types=[
        pltpu.VMEM((_DCH, _CW), jnp.int32),
        pltpu.VMEM((_CW,), jnp.float32),
        pltpu.VMEM((_RPT,), jnp.float32),
        pltpu.VMEM_SHARED((_NPAD,), jnp.float32),
    ],
)
def _deg_kernel(src_hbm, inv_hbm, idx_v, ones_v, val_v, deg_sh):
    c = lax.axis_index("c")
    s = lax.axis_index("s")
    pltpu.sync_copy(src_hbm.at[s], idx_v)

    for q in range(_CW // 16):
        ones_v[pl.ds(q * 16, 16)] = jnp.full((16,), 1.0, jnp.float32)

    def _zero_val(j, _):
        val_v[pl.ds(j * 16, 16)] = jnp.zeros((16,), jnp.float32)
        return _

    lax.fori_loop(0, _RPT // 16, _zero_val, None)
    pltpu.sync_copy(val_v, deg_sh.at[pl.ds(s * _RPT, _RPT)])
    plsc.subcore_barrier()

    def _scat(i, _):
        pltpu.sync_copy(ones_v, deg_sh.at[idx_v.at[i]], add=True)
        return _

    lax.fori_loop(0, _DCH, _scat, None)
    plsc.subcore_barrier()
    pltpu.sync_copy(deg_sh.at[pl.ds(s * _RPT, _RPT)], val_v)

    def _inv(j, _):
        v = val_v[pl.ds(j * 16, 16)]
        val_v[pl.ds(j * 16, 16)] = 1.0 / jnp.maximum(v, 1.0)
        return _

    lax.fori_loop(0, _RPT // 16, _inv, None)

    @pl.when(c == 0)
    def _():
        pltpu.sync_copy(val_v, inv_hbm.at[pl.ds(s * _RPT, _RPT)])


@functools.partial(
    pl.kernel,
    out_type=jax.ShapeDtypeStruct((_NC, _NPAD, _D), jnp.float32),
    mesh=_mesh,
    scratch_types=[
        pltpu.VMEM((_NCH, _CW), jnp.int32),
        pltpu.VMEM((_NCH, _CW), jnp.int32),
        pltpu.VMEM((_CW, _D), jnp.float32),
        pltpu.VMEM_SHARED((_NPAD, _D), jnp.float32),
    ],
)
def _agg_kernel(tbl_hbm, src_hbm, dst_hbm, out_hbm, sidx_v, didx_v, buf_v,
                acc_sh):
    c = lax.axis_index("c")
    s = lax.axis_index("s")
    w = s * _NC + c
    pltpu.sync_copy(src_hbm.at[w], sidx_v)
    pltpu.sync_copy(dst_hbm.at[w], didx_v)

    def _zero_buf(i, _):
        for q in range(_D // 16):
            buf_v[i, pl.ds(q * 16, 16)] = jnp.zeros((16,), jnp.float32)
        return _

    lax.fori_loop(0, _CW, _zero_buf, None)

    def _zero_acc(k, _):
        pltpu.sync_copy(buf_v, acc_sh.at[pl.ds(s * _RPT + k * _CW, _CW)])
        return _

    lax.fori_loop(0, _RPT // _CW, _zero_acc, None)
    plsc.subcore_barrier()

    def _step(i, _):
        pltpu.sync_copy(tbl_hbm.at[sidx_v.at[i]], buf_v)
        pltpu.sync_copy(buf_v, acc_sh.at[didx_v.at[i]], add=True)
        return _

    lax.fori_loop(0, _NCH, _step, None)
    plsc.subcore_barrier()
    pltpu.sync_copy(acc_sh.at[pl.ds(s * _RPT, _RPT)],
                    out_hbm.at[c, pl.ds(s * _RPT, _RPT)])


_BR = 400
_HIGH = lax.Precision.HIGHEST


def _scale_body(x_ref, inv_ref, o_ref):
    o_ref[...] = x_ref[...] * inv_ref[...]


_scale = pl.pallas_call(
    _scale_body,
    grid=(_N // _BR,),
    in_specs=[
        pl.BlockSpec((_BR, _D), lambda i: (i, 0)),
        pl.BlockSpec((_BR, 1), lambda i: (i, 0)),
    ],
    out_specs=pl.BlockSpec((_BR, _D), lambda i: (i, 0)),
    out_shape=jax.ShapeDtypeStruct((_N, _D), jnp.float32),
)


def _mm1_body(p_ref, w_ref, b_ref, inv_ref, o_ref):
    a = p_ref[0] + p_ref[1]
    h = jnp.dot(a, w_ref[...], preferred_element_type=jnp.float32,
                precision=_HIGH) + b_ref[...]
    o_ref[...] = jnp.maximum(h, 0.0) * inv_ref[...]


_mm1 = pl.pallas_call(
    _mm1_body,
    grid=(_N // _BR,),
    in_specs=[
        pl.BlockSpec((_NC, _BR, _D), lambda i: (0, i, 0)),
        pl.BlockSpec((_D, _D), lambda i: (0, 0)),
        pl.BlockSpec((1, _D), lambda i: (0, 0)),
        pl.BlockSpec((_BR, 1), lambda i: (i, 0)),
    ],
    out_specs=pl.BlockSpec((_BR, _D), lambda i: (i, 0)),
    out_shape=jax.ShapeDtypeStruct((_N, _D), jnp.float32),
)


def _fin_body(p_ref, w1_ref, b1_ref, wp_ref, bp_ref, wv_ref, bv_ref,
              pi_ref, v_ref, acc_ref):
    i = pl.program_id(0)
    a = p_ref[0] + p_ref[1]
    h2 = jnp.dot(a, w1_ref[...], preferred_element_type=jnp.float32,
                 precision=_HIGH) + b1_ref[...]
    pi_ref[...] = jnp.dot(h2, wp_ref[...], preferred_element_type=jnp.float32,
                          precision=_HIGH) + bp_ref[...]
    colsum = jnp.sum(a, axis=0, keepdims=True)

    @pl.when(i == 0)
    def _():
        acc_ref[...] = colsum

    @pl.when(i > 0)
    def _():
        acc_ref[...] = acc_ref[...] + colsum

    @pl.when(i == _N // _BR - 1)
    def _():
        m = acc_ref[...] * (1.0 / _N)
        h2m = jnp.dot(m, w1_ref[...], preferred_element_type=jnp.float32,
                      precision=_HIGH) + b1_ref[...]
        v_ref[...] = jnp.dot(h2m, wv_ref[...],
                             preferred_element_type=jnp.float32,
                             precision=_HIGH) + bv_ref[...]


_fin = pl.pallas_call(
    _fin_body,
    grid=(_N // _BR,),
    in_specs=[
        pl.BlockSpec((_NC, _BR, _D), lambda i: (0, i, 0)),
        pl.BlockSpec((_D, _D), lambda i: (0, 0)),
        pl.BlockSpec((1, _D), lambda i: (0, 0)),
        pl.BlockSpec((_D, 1), lambda i: (0, 0)),
        pl.BlockSpec((1, 1), lambda i: (0, 0)),
        pl.BlockSpec((_D, 1), lambda i: (0, 0)),
        pl.BlockSpec((1, 1), lambda i: (0, 0)),
    ],
    out_specs=[
        pl.BlockSpec((_BR, 1), lambda i: (i, 0)),
        pl.BlockSpec((1, 1), lambda i: (0, 0)),
    ],
    out_shape=[
        jax.ShapeDtypeStruct((_N, 1), jnp.float32),
        jax.ShapeDtypeStruct((1, 1), jnp.float32),
    ],
    scratch_shapes=[pltpu.VMEM((1, _D), jnp.float32)],
)


def _pad_slabs(a, nslab, per_slab, per_slab_pad, pad_vals):
    a2 = a.reshape(nslab, per_slab)
    pad = pad_vals.reshape(nslab, per_slab_pad - per_slab)
    return jnp.concatenate([a2, pad], axis=1).reshape(nslab, -1, _CW)


def kernel(x, edge_index, W0, b0, W1, b1, Wp, bp, Wv, bv):
    src = edge_index[0]
    dst = edge_index[1]

    npad_w = _EPWP - _EPW
    pad_src_w = (jnp.arange(_NW * npad_w, dtype=jnp.int32) * 97) % _N
    pad_dst_w = _N + (jnp.arange(_NW * npad_w, dtype=jnp.int32) % (_NPAD - _N))
    npad_t = _EPTP - _EPT
    pad_src_t = _N + (jnp.arange(_NS * npad_t, dtype=jnp.int32) % (_NPAD - _N))

    src3 = _pad_slabs(src, _NW, _EPW, _EPWP, pad_src_w)
    dst3 = _pad_slabs(dst, _NW, _EPW, _EPWP, pad_dst_w)
    src_deg = _pad_slabs(src, _NS, _EPT, _EPTP, pad_src_t)

    inv_pad = _deg_kernel(src_deg)
    inv2 = inv_pad[:_N].reshape(_N, 1)

    xn = _scale(x, inv2)
    p1 = _agg_kernel(xn, src3, dst3)
    h1n = _mm1(p1, W0, b0.reshape(1, _D), inv2)
    p2 = _agg_kernel(h1n, src3, dst3)
    PI, V = _fin(p2, W1, b1.reshape(1, _D), Wp, bp.reshape(1, 1),
                 Wv, bv.reshape(1, 1))
    return (PI, V)


# speedup vs baseline: 16.6484x; 1.2580x over previous
import functools

import jax
import jax.numpy as jnp
from jax import lax
from jax.experimental import pallas as pl
from jax.experimental.pallas import tpu as pltpu
from jax.experimental.pallas import tpu_sc as plsc

_N = 10000
_E = 320000
_D = 128
_NPAD = 10240
_NC = 2
_NS = 16
_NW = _NC * _NS
_RPT = _NPAD // _NS
_CW = 128
_EPW = _E // _NW
_NCH = -(-_EPW // _CW)
_EPWP = _NCH * _CW
_EPT = _E // _NS
_DCH = -(-_EPT // _CW)
_EPTP = _DCH * _CW

_mesh = plsc.VectorSubcoreMesh(core_axis_name="c", subcore_axis_name="s")


@functools.partial(
    pl.kernel,
    out_type=[jax.ShapeDtypeStruct((_NPAD,), jnp.float32),
              jax.ShapeDtypeStruct((_NPAD,), jnp.float32)],
    mesh=_mesh,
    scratch_types=[
        pltpu.VMEM((_DCH, _CW), jnp.int32),
        pltpu.VMEM((_CW,), jnp.float32),
        pltpu.VMEM((_RPT,), jnp.float32),
        pltpu.VMEM_SHARED((_NPAD,), jnp.float32),
    ],
)
def _deg_kernel(src_hbm, inv_hbm, deg_hbm, idx_v, ones_v, val_v, deg_sh):
    c = lax.axis_index("c")
    s = lax.axis_index("s")
    pltpu.sync_copy(src_hbm.at[s], idx_v)

    for q in range(_CW // 16):
        ones_v[pl.ds(q * 16, 16)] = jnp.full((16,), 1.0, jnp.float32)

    def _zero_val(j, _):
        val_v[pl.ds(j * 16, 16)] = jnp.zeros((16,), jnp.float32)
        return _

    lax.fori_loop(0, _RPT // 16, _zero_val, None)
    pltpu.sync_copy(val_v, deg_sh.at[pl.ds(s * _RPT, _RPT)])
    plsc.subcore_barrier()

    def _scat(i, _):
        pltpu.sync_copy(ones_v, deg_sh.at[idx_v.at[i]], add=True)
        return _

    lax.fori_loop(0, _DCH, _scat, None)
    plsc.subcore_barrier()
    pltpu.sync_copy(deg_sh.at[pl.ds(s * _RPT, _RPT)], val_v)

    @pl.when(c == 0)
    def _():
        pltpu.sync_copy(val_v, deg_hbm.at[pl.ds(s * _RPT, _RPT)])

    def _inv(j, _):
        v = val_v[pl.ds(j * 16, 16)]
        val_v[pl.ds(j * 16, 16)] = 1.0 / jnp.maximum(v, 1.0)
        return _

    lax.fori_loop(0, _RPT // 16, _inv, None)

    @pl.when(c == 0)
    def _():
        pltpu.sync_copy(val_v, inv_hbm.at[pl.ds(s * _RPT, _RPT)])


@functools.partial(
    pl.kernel,
    out_type=jax.ShapeDtypeStruct((_NC, _NPAD, _D), jnp.float32),
    mesh=_mesh,
    scratch_types=[
        pltpu.VMEM((_NCH, _CW), jnp.int32),
        pltpu.VMEM((_NCH, _CW), jnp.int32),
        pltpu.VMEM((_CW, _D), jnp.float32),
        pltpu.VMEM_SHARED((_NPAD, _D), jnp.float32),
    ],
)
def _agg_kernel(tbl_hbm, src_hbm, dst_hbm, out_hbm, sidx_v, didx_v, buf_v,
                acc_sh):
    c = lax.axis_index("c")
    s = lax.axis_index("s")
    w = s * _NC + c
    pltpu.sync_copy(src_hbm.at[w], sidx_v)
    pltpu.sync_copy(dst_hbm.at[w], didx_v)

    def _zero_buf(i, _):
        for q in range(_D // 16):
            buf_v[i, pl.ds(q * 16, 16)] = jnp.zeros((16,), jnp.float32)
        return _

    lax.fori_loop(0, _CW, _zero_buf, None)

    def _zero_acc(k, _):
        pltpu.sync_copy(buf_v, acc_sh.at[pl.ds(s * _RPT + k * _CW, _CW)])
        return _

    lax.fori_loop(0, _RPT // _CW, _zero_acc, None)
    plsc.subcore_barrier()

    def _step(i, _):
        pltpu.sync_copy(tbl_hbm.at[sidx_v.at[i]], buf_v)
        pltpu.sync_copy(buf_v, acc_sh.at[didx_v.at[i]], add=True)
        return _

    lax.fori_loop(0, _NCH, _step, None)
    plsc.subcore_barrier()
    pltpu.sync_copy(acc_sh.at[pl.ds(s * _RPT, _RPT)],
                    out_hbm.at[c, pl.ds(s * _RPT, _RPT)])


@functools.partial(
    pl.kernel,
    out_type=jax.ShapeDtypeStruct((_NC, _NPAD), jnp.float32),
    mesh=_mesh,
    scratch_types=[
        pltpu.VMEM((_NCH, _CW), jnp.int32),
        pltpu.VMEM((_NCH, _CW), jnp.int32),
        pltpu.VMEM((_CW,), jnp.float32),
        pltpu.VMEM((_RPT,), jnp.float32),
        pltpu.VMEM_SHARED((_NPAD,), jnp.float32),
    ],
)
def _aggz_kernel(z_hbm, src_hbm, dst_hbm, out_hbm, sidx_v, didx_v, zbuf_v,
                 val_v, acc_sh):
    c = lax.axis_index("c")
    s = lax.axis_index("s")
    w = s * _NC + c
    pltpu.sync_copy(src_hbm.at[w], sidx_v)
    pltpu.sync_copy(dst_hbm.at[w], didx_v)

    def _zero_val(j, _):
        val_v[pl.ds(j * 16, 16)] = jnp.zeros((16,), jnp.float32)
        return _

    lax.fori_loop(0, _RPT // 16, _zero_val, None)
    pltpu.sync_copy(val_v, acc_sh.at[pl.ds(s * _RPT, _RPT)])
    plsc.subcore_barrier()

    def _step(i, _):
        pltpu.sync_copy(z_hbm.at[sidx_v.at[i]], zbuf_v)
        pltpu.sync_copy(zbuf_v, acc_sh.at[didx_v.at[i]], add=True)
        return _

    lax.fori_loop(0, _NCH, _step, None)
    plsc.subcore_barrier()
    pltpu.sync_copy(acc_sh.at[pl.ds(s * _RPT, _RPT)],
                    out_hbm.at[c, pl.ds(s * _RPT, _RPT)])


_BR = 400
_HIGH = lax.Precision.HIGHEST


def _scale_body(x_ref, inv_ref, o_ref):
    o_ref[...] = x_ref[...] * inv_ref[...]


_scale = pl.pallas_call(
    _scale_body,
    grid=(_N // _BR,),
    in_specs=[
        pl.BlockSpec((_BR, _D), lambda i: (i, 0)),
        pl.BlockSpec((_BR, 1), lambda i: (i, 0)),
    ],
    out_specs=pl.BlockSpec((_BR, _D), lambda i: (i, 0)),
    out_shape=jax.ShapeDtypeStruct((_N, _D), jnp.float32),
)


def _mm2_body(p_ref, w0_ref, b0_ref, inv_ref, w1_ref, wp_ref, deg_ref,
              z_ref, ws_ref, acc_ref):
    i = pl.program_id(0)
    a = p_ref[0] + p_ref[1]
    h = jnp.dot(a, w0_ref[...], preferred_element_type=jnp.float32,
                precision=_HIGH) + b0_ref[...]
    h1n = jnp.maximum(h, 0.0) * inv_ref[...]
    w1p = jnp.dot(w1_ref[...], wp_ref[...], preferred_element_type=jnp.float32,
                  precision=_HIGH)
    z_ref[...] = jnp.dot(h1n, w1p, preferred_element_type=jnp.float32,
                         precision=_HIGH)
    wcol = jnp.sum(h1n * deg_ref[...], axis=0, keepdims=True)

    @pl.when(i == 0)
    def _():
        acc_ref[...] = wcol

    @pl.when(i > 0)
    def _():
        acc_ref[...] = acc_ref[...] + wcol

    @pl.when(i == _N // _BR - 1)
    def _():
        ws_ref[...] = acc_ref[...]


_mm2 = pl.pallas_call(
    _mm2_body,
    grid=(_N // _BR,),
    in_specs=[
        pl.BlockSpec((_NC, _BR, _D), lambda i: (0, i, 0)),
        pl.BlockSpec((_D, _D), lambda i: (0, 0)),
        pl.BlockSpec((1, _D), lambda i: (0, 0)),
        pl.BlockSpec((_BR, 1), lambda i: (i, 0)),
        pl.BlockSpec((_D, _D), lambda i: (0, 0)),
        pl.BlockSpec((_D, 1), lambda i: (0, 0)),
        pl.BlockSpec((_BR, 1), lambda i: (i, 0)),
    ],
    out_specs=[
        pl.BlockSpec((_BR, 1), lambda i: (i, 0)),
        pl.BlockSpec((1, _D), lambda i: (0, 0)),
    ],
    out_shape=[
        jax.ShapeDtypeStruct((_N, 1), jnp.float32),
        jax.ShapeDtypeStruct((1, _D), jnp.float32),
    ],
    scratch_shapes=[pltpu.VMEM((1, _D), jnp.float32)],
)

_BZ = 512


def _finz_body(az_ref, ws_ref, w1_ref, b1_ref, wp_ref, bp_ref, wv_ref, bv_ref,
               pi_ref, v_ref):
    i = pl.program_id(0)
    az = az_ref[0] + az_ref[1]
    c0 = jnp.dot(b1_ref[...], wp_ref[...], preferred_element_type=jnp.float32,
                 precision=_HIGH) + bp_ref[...]
    pi_ref[...] = az[:, None] + c0

    @pl.when(i == 0)
    def _():
        m = ws_ref[...] * (1.0 / _N)
        h2m = jnp.dot(m, w1_ref[...], preferred_element_type=jnp.float32,
                      precision=_HIGH) + b1_ref[...]
        v_ref[...] = jnp.dot(h2m, wv_ref[...],
                             preferred_element_type=jnp.float32,
                             precision=_HIGH) + bv_ref[...]


_finz = pl.pallas_call(
    _finz_body,
    grid=(_NPAD // _BZ,),
    in_specs=[
        pl.BlockSpec((_NC, _BZ), lambda i: (0, i)),
        pl.BlockSpec((1, _D), lambda i: (0, 0)),
        pl.BlockSpec((_D, _D), lambda i: (0, 0)),
        pl.BlockSpec((1, _D), lambda i: (0, 0)),
        pl.BlockSpec((_D, 1), lambda i: (0, 0)),
        pl.BlockSpec((1, 1), lambda i: (0, 0)),
        pl.BlockSpec((_D, 1), lambda i: (0, 0)),
        pl.BlockSpec((1, 1), lambda i: (0, 0)),
    ],
    out_specs=[
        pl.BlockSpec((_BZ, 1), lambda i: (i, 0)),
        pl.BlockSpec((1, 1), lambda i: (0, 0)),
    ],
    out_shape=[
        jax.ShapeDtypeStruct((_NPAD, 1), jnp.float32),
        jax.ShapeDtypeStruct((1, 1), jnp.float32),
    ],
)


def _pad_slabs(a, nslab, per_slab, per_slab_pad, pad_vals):
    a2 = a.reshape(nslab, per_slab)
    pad = pad_vals.reshape(nslab, per_slab_pad - per_slab)
    return jnp.concatenate([a2, pad], axis=1).reshape(nslab, -1, _CW)


def kernel(x, edge_index, W0, b0, W1, b1, Wp, bp, Wv, bv):
    src = edge_index[0]
    dst = edge_index[1]

    npad_w = _EPWP - _EPW
    pad_src_w = (jnp.arange(_NW * npad_w, dtype=jnp.int32) * 97) % _N
    pad_dst_w = _N + (jnp.arange(_NW * npad_w, dtype=jnp.int32) % (_NPAD - _N))
    npad_t = _EPTP - _EPT
    pad_src_t = _N + (jnp.arange(_NS * npad_t, dtype=jnp.int32) % (_NPAD - _N))

    src3 = _pad_slabs(src, _NW, _EPW, _EPWP, pad_src_w)
    dst3 = _pad_slabs(dst, _NW, _EPW, _EPWP, pad_dst_w)
    src_deg = _pad_slabs(src, _NS, _EPT, _EPTP, pad_src_t)

    inv_pad, deg_pad = _deg_kernel(src_deg)
    inv2 = inv_pad[:_N].reshape(_N, 1)
    deg2 = deg_pad[:_N].reshape(_N, 1)

    xn = _scale(x, inv2)
    p1 = _agg_kernel(xn, src3, dst3)
    z2, wsum = _mm2(p1, W0, b0.reshape(1, _D), inv2, W1, Wp, deg2)
    azp = _aggz_kernel(z2.reshape(_N), src3, dst3)
    PI_pad, V = _finz(azp, wsum, W1, b1.reshape(1, _D), Wp, bp.reshape(1, 1),
                      Wv, bv.reshape(1, 1))
    return (PI_pad[:_N], V)


# speedup vs baseline: 16.9981x; 1.0210x over previous
import functools

import jax
import jax.numpy as jnp
from jax import lax
from jax.experimental import pallas as pl
from jax.experimental.pallas import tpu as pltpu
from jax.experimental.pallas import tpu_sc as plsc

_N = 10000
_E = 320000
_D = 128
_NPAD = 10240
_NC = 2
_NS = 16
_NW = _NC * _NS
_RPT = _NPAD // _NS
_CW = 128
_EPW = _E // _NW
_NCH = -(-_EPW // _CW)
_EPWP = _NCH * _CW
_EPT = _E // _NS
_DCH = -(-_EPT // _CW)
_EPTP = _DCH * _CW

_mesh = plsc.VectorSubcoreMesh(core_axis_name="c", subcore_axis_name="s")


@functools.partial(
    pl.kernel,
    out_type=[jax.ShapeDtypeStruct((_NPAD,), jnp.float32),
              jax.ShapeDtypeStruct((_NPAD,), jnp.float32)],
    mesh=_mesh,
    scratch_types=[
        pltpu.VMEM((_DCH, _CW), jnp.int32),
        pltpu.VMEM((_CW,), jnp.float32),
        pltpu.VMEM((_RPT,), jnp.float32),
        pltpu.VMEM_SHARED((_NPAD,), jnp.float32),
    ],
)
def _deg_kernel(src_hbm, inv_hbm, deg_hbm, idx_v, ones_v, val_v, deg_sh):
    c = lax.axis_index("c")
    s = lax.axis_index("s")
    pltpu.sync_copy(src_hbm.at[s], idx_v)

    for q in range(_CW // 16):
        ones_v[pl.ds(q * 16, 16)] = jnp.full((16,), 1.0, jnp.float32)

    def _zero_val(j, _):
        val_v[pl.ds(j * 16, 16)] = jnp.zeros((16,), jnp.float32)
        return _

    lax.fori_loop(0, _RPT // 16, _zero_val, None)
    pltpu.sync_copy(val_v, deg_sh.at[pl.ds(s * _RPT, _RPT)])
    plsc.subcore_barrier()

    def _scat(i, _):
        pltpu.sync_copy(ones_v, deg_sh.at[idx_v.at[i]], add=True)
        return _

    lax.fori_loop(0, _DCH, _scat, None)
    plsc.subcore_barrier()
    pltpu.sync_copy(deg_sh.at[pl.ds(s * _RPT, _RPT)], val_v)

    @pl.when(c == 0)
    def _():
        pltpu.sync_copy(val_v, deg_hbm.at[pl.ds(s * _RPT, _RPT)])

    def _inv(j, _):
        v = val_v[pl.ds(j * 16, 16)]
        val_v[pl.ds(j * 16, 16)] = 1.0 / jnp.maximum(v, 1.0)
        return _

    lax.fori_loop(0, _RPT // 16, _inv, None)

    @pl.when(c == 0)
    def _():
        pltpu.sync_copy(val_v, inv_hbm.at[pl.ds(s * _RPT, _RPT)])


@functools.partial(
    pl.kernel,
    out_type=jax.ShapeDtypeStruct((_NC, _NPAD, _D), jnp.float32),
    mesh=_mesh,
    scratch_types=[
        pltpu.VMEM((_NCH, _CW), jnp.int32),
        pltpu.VMEM((_NCH, _CW), jnp.int32),
        pltpu.VMEM((_CW, _D), jnp.float32),
        pltpu.VMEM_SHARED((_NPAD, _D), jnp.float32),
    ],
)
def _agg_kernel(tbl_hbm, src_hbm, dst_hbm, out_hbm, sidx_v, didx_v, buf_v,
                acc_sh):
    c = lax.axis_index("c")
    s = lax.axis_index("s")
    w = s * _NC + c
    pltpu.sync_copy(src_hbm.at[w], sidx_v)
    pltpu.sync_copy(dst_hbm.at[w], didx_v)

    def _zero_buf(i, _):
        for q in range(_D // 16):
            buf_v[i, pl.ds(q * 16, 16)] = jnp.zeros((16,), jnp.float32)
        return _

    lax.fori_loop(0, _CW, _zero_buf, None)

    def _zero_acc(k, _):
        pltpu.sync_copy(buf_v, acc_sh.at[pl.ds(s * _RPT + k * _CW, _CW)])
        return _

    lax.fori_loop(0, _RPT // _CW, _zero_acc, None)
    plsc.subcore_barrier()

    def _step(i, _):
        pltpu.sync_copy(tbl_hbm.at[sidx_v.at[i]], buf_v)
        pltpu.sync_copy(buf_v, acc_sh.at[didx_v.at[i]], add=True)
        return _

    lax.fori_loop(0, _NCH, _step, None)
    plsc.subcore_barrier()
    pltpu.sync_copy(acc_sh.at[pl.ds(s * _RPT, _RPT)],
                    out_hbm.at[c, pl.ds(s * _RPT, _RPT)])


@functools.partial(
    pl.kernel,
    out_type=jax.ShapeDtypeStruct((_NC, _NPAD), jnp.float32),
    mesh=_mesh,
    scratch_types=[
        pltpu.VMEM((_NCH, _CW), jnp.int32),
        pltpu.VMEM((_NCH, _CW), jnp.int32),
        pltpu.VMEM((2, _CW), jnp.float32),
        pltpu.VMEM((_RPT,), jnp.float32),
        pltpu.SemaphoreType.DMA,
        pltpu.VMEM_SHARED((_NPAD,), jnp.float32),
    ],
)
def _aggz_kernel(z_hbm, src_hbm, dst_hbm, out_hbm, sidx_v, didx_v, zbuf_v,
                 val_v, gsem, acc_sh):
    c = lax.axis_index("c")
    s = lax.axis_index("s")
    w = s * _NC + c
    pltpu.sync_copy(src_hbm.at[w], sidx_v)
    pltpu.sync_copy(dst_hbm.at[w], didx_v)

    def _zero_val(j, _):
        val_v[pl.ds(j * 16, 16)] = jnp.zeros((16,), jnp.float32)
        return _

    lax.fori_loop(0, _RPT // 16, _zero_val, None)
    pltpu.sync_copy(val_v, acc_sh.at[pl.ds(s * _RPT, _RPT)])
    plsc.subcore_barrier()

    pltpu.async_copy(z_hbm.at[sidx_v.at[0]], zbuf_v.at[0], gsem)

    def _step(i, _):
        b = lax.rem(i, 2)
        pltpu.make_async_copy(z_hbm.at[sidx_v.at[i]], zbuf_v.at[b],
                              gsem).wait()

        @pl.when(i + 1 < _NCH)
        def _():
            pltpu.async_copy(z_hbm.at[sidx_v.at[i + 1]], zbuf_v.at[1 - b],
                             gsem)

        pltpu.sync_copy(zbuf_v.at[b], acc_sh.at[didx_v.at[i]], add=True)
        return _

    lax.fori_loop(0, _NCH, _step, None)
    plsc.subcore_barrier()
    pltpu.sync_copy(acc_sh.at[pl.ds(s * _RPT, _RPT)],
                    out_hbm.at[c, pl.ds(s * _RPT, _RPT)])


_BR = 400
_HIGH = lax.Precision.HIGHEST


def _scale_body(x_ref, inv_ref, o_ref):
    o_ref[...] = x_ref[...] * inv_ref[...]


_scale = pl.pallas_call(
    _scale_body,
    grid=(_N // _BR,),
    in_specs=[
        pl.BlockSpec((_BR, _D), lambda i: (i, 0)),
        pl.BlockSpec((_BR, 1), lambda i: (i, 0)),
    ],
    out_specs=pl.BlockSpec((_BR, _D), lambda i: (i, 0)),
    out_shape=jax.ShapeDtypeStruct((_N, _D), jnp.float32),
)


def _mm2_body(p_ref, w0_ref, b0_ref, inv_ref, w1_ref, wp_ref, deg_ref,
              z_ref, ws_ref, acc_ref):
    i = pl.program_id(0)
    a = p_ref[0] + p_ref[1]
    h = jnp.dot(a, w0_ref[...], preferred_element_type=jnp.float32,
                precision=_HIGH) + b0_ref[...]
    h1n = jnp.maximum(h, 0.0) * inv_ref[...]
    w1p = jnp.dot(w1_ref[...], wp_ref[...], preferred_element_type=jnp.float32,
                  precision=_HIGH)
    z_ref[...] = jnp.dot(h1n, w1p, preferred_element_type=jnp.float32,
                         precision=_HIGH)
    wcol = jnp.sum(h1n * deg_ref[...], axis=0, keepdims=True)

    @pl.when(i == 0)
    def _():
        acc_ref[...] = wcol

    @pl.when(i > 0)
    def _():
        acc_ref[...] = acc_ref[...] + wcol

    @pl.when(i == _N // _BR - 1)
    def _():
        ws_ref[...] = acc_ref[...]


_mm2 = pl.pallas_call(
    _mm2_body,
    grid=(_N // _BR,),
    in_specs=[
        pl.BlockSpec((_NC, _BR, _D), lambda i: (0, i, 0)),
        pl.BlockSpec((_D, _D), lambda i: (0, 0)),
        pl.BlockSpec((1, _D), lambda i: (0, 0)),
        pl.BlockSpec((_BR, 1), lambda i: (i, 0)),
        pl.BlockSpec((_D, _D), lambda i: (0, 0)),
        pl.BlockSpec((_D, 1), lambda i: (0, 0)),
        pl.BlockSpec((_BR, 1), lambda i: (i, 0)),
    ],
    out_specs=[
        pl.BlockSpec((_BR, 1), lambda i: (i, 0)),
        pl.BlockSpec((1, _D), lambda i: (0, 0)),
    ],
    out_shape=[
        jax.ShapeDtypeStruct((_N, 1), jnp.float32),
        jax.ShapeDtypeStruct((1, _D), jnp.float32),
    ],
    scratch_shapes=[pltpu.VMEM((1, _D), jnp.float32)],
)

_BZ = 512


def _finz_body(az_ref, ws_ref, w1_ref, b1_ref, wp_ref, bp_ref, wv_ref, bv_ref,
               pi_ref, v_ref):
    i = pl.program_id(0)
    az = az_ref[0] + az_ref[1]
    c0 = jnp.dot(b1_ref[...], wp_ref[...], preferred_element_type=jnp.float32,
                 precision=_HIGH) + bp_ref[...]
    pi_ref[...] = az[:, None] + c0

    @pl.when(i == 0)
    def _():
        m = ws_ref[...] * (1.0 / _N)
        h2m = jnp.dot(m, w1_ref[...], preferred_element_type=jnp.float32,
                      precision=_HIGH) + b1_ref[...]
        v_ref[...] = jnp.dot(h2m, wv_ref[...],
                             preferred_element_type=jnp.float32,
                             precision=_HIGH) + bv_ref[...]


_finz = pl.pallas_call(
    _finz_body,
    grid=(_NPAD // _BZ,),
    in_specs=[
        pl.BlockSpec((_NC, _BZ), lambda i: (0, i)),
        pl.BlockSpec((1, _D), lambda i: (0, 0)),
        pl.BlockSpec((_D, _D), lambda i: (0, 0)),
        pl.BlockSpec((1, _D), lambda i: (0, 0)),
        pl.BlockSpec((_D, 1), lambda i: (0, 0)),
        pl.BlockSpec((1, 1), lambda i: (0, 0)),
        pl.BlockSpec((_D, 1), lambda i: (0, 0)),
        pl.BlockSpec((1, 1), lambda i: (0, 0)),
    ],
    out_specs=[
        pl.BlockSpec((_BZ, 1), lambda i: (i, 0)),
        pl.BlockSpec((1, 1), lambda i: (0, 0)),
    ],
    out_shape=[
        jax.ShapeDtypeStruct((_NPAD, 1), jnp.float32),
        jax.ShapeDtypeStruct((1, 1), jnp.float32),
    ],
)


def _pad_slabs(a, nslab, per_slab, per_slab_pad, pad_vals):
    a2 = a.reshape(nslab, per_slab)
    pad = pad_vals.reshape(nslab, per_slab_pad - per_slab)
    return jnp.concatenate([a2, pad], axis=1).reshape(nslab, -1, _CW)


def kernel(x, edge_index, W0, b0, W1, b1, Wp, bp, Wv, bv):
    src = edge_index[0]
    dst = edge_index[1]

    npad_w = _EPWP - _EPW
    pad_src_w = (jnp.arange(_NW * npad_w, dtype=jnp.int32) * 97) % _N
    pad_dst_w = _N + (jnp.arange(_NW * npad_w, dtype=jnp.int32) % (_NPAD - _N))
    npad_t = _EPTP - _EPT
    pad_src_t = _N + (jnp.arange(_NS * npad_t, dtype=jnp.int32) % (_NPAD - _N))

    src3 = _pad_slabs(src, _NW, _EPW, _EPWP, pad_src_w)
    dst3 = _pad_slabs(dst, _NW, _EPW, _EPWP, pad_dst_w)
    src_deg = _pad_slabs(src, _NS, _EPT, _EPTP, pad_src_t)

    inv_pad, deg_pad = _deg_kernel(src_deg)
    inv2 = inv_pad[:_N].reshape(_N, 1)
    deg2 = deg_pad[:_N].reshape(_N, 1)

    xn = _scale(x, inv2)
    p1 = _agg_kernel(xn, src3, dst3)
    z2, wsum = _mm2(p1, W0, b0.reshape(1, _D), inv2, W1, Wp, deg2)
    azp = _aggz_kernel(z2.reshape(_N), src3, dst3)
    PI_pad, V = _finz(azp, wsum, W1, b1.reshape(1, _D), Wp, bp.reshape(1, 1),
                      Wv, bv.reshape(1, 1))
    return (PI_pad[:_N], V)
